# baseline (speedup 1.0000x reference)
import jax
import jax.numpy as jnp
from jax.experimental import pallas as pl

EMB = 4096
NE = 64
K = 8
NT = 8192
M_BLK = 512
CH = 64


def _router_block(x_ref, w_ref, probs_ref, idx_ref, scores_ref):
    x = x_ref[...]
    w = w_ref[...]
    scores = jax.lax.dot_general(
        x, w, (((1,), (1,)), ((), ())), preferred_element_type=jnp.float32
    )
    scores_ref[...] = scores

    cols = jax.lax.broadcasted_iota(jnp.int32, (CH, NE), 1)
    tie = jnp.int32(NE - 1) - cols
    neg_min = jnp.int32(-(2**31))
    for c in range(M_BLK // CH):
        s = jax.lax.slice(scores, (c * CH, 0), ((c + 1) * CH, NE))
        i = jax.lax.bitcast_convert_type(s, jnp.int32)
        o = i ^ jax.lax.shift_right_arithmetic(i, 31).astype(jnp.int32) & jnp.int32(0x7FFFFFFF)
        key = (o & jnp.int32(~(NE - 1))) | tie
        tops = []
        for _ in range(K):
            m = jnp.max(key, axis=1, keepdims=True)
            tops.append(m)
            key = jnp.where(key == m, neg_min, key)
        tk = jnp.concatenate(tops, axis=1)
        top_idx = jnp.int32(NE - 1) - (tk & jnp.int32(NE - 1))
        ov = tk & jnp.int32(~(NE - 1))
        iv = ov ^ jax.lax.shift_right_arithmetic(ov, 31).astype(jnp.int32) & jnp.int32(0x7FFFFFFF)
        vals = jax.lax.bitcast_convert_type(iv, jnp.float32)
        e = jnp.exp(vals - vals[:, 0:1])
        probs = e / jnp.sum(e, axis=1, keepdims=True)
        probs_ref[c * CH:(c + 1) * CH, :] = probs
        idx_ref[c * CH:(c + 1) * CH, :] = top_idx


@jax.jit
def kernel(x, W_gate):
    grid = (NT // M_BLK,)
    probs, idx, scores = pl.pallas_call(
        _router_block,
        grid=grid,
        in_specs=[
            pl.BlockSpec((M_BLK, EMB), lambda i: (i, 0)),
            pl.BlockSpec((NE, EMB), lambda i: (0, 0)),
        ],
        out_specs=[
            pl.BlockSpec((M_BLK, K), lambda i: (i, 0)),
            pl.BlockSpec((M_BLK, K), lambda i: (i, 0)),
            pl.BlockSpec((M_BLK, NE), lambda i: (i, 0)),
        ],
        out_shape=[
            jax.ShapeDtypeStruct((NT, K), jnp.float32),
            jax.ShapeDtypeStruct((NT, K), jnp.int32),
            jax.ShapeDtypeStruct((NT, NE), jnp.float32),
        ],
    )(x, W_gate)
    return (probs, idx, scores)


# speedup vs baseline: 1.1048x; 1.1048x over previous
import jax
import jax.numpy as jnp
from jax.experimental import pallas as pl

EMB = 4096
NE = 64
K = 8
NT = 8192
M_BLK = 512
CH = 64


def _router_block(x_ref, w_ref, probs_ref, idx_ref, scores_ref):
    x = x_ref[...]
    w = w_ref[...]
    scores = jax.lax.dot_general(
        x, w, (((1,), (1,)), ((), ())), preferred_element_type=jnp.float32
    )
    scores_ref[...] = scores

    cols63 = jnp.int32(NE - 1) - jax.lax.broadcasted_iota(jnp.int32, (CH, NE), 1)
    m6 = jnp.int32(NE - 1)
    neg_inf = jnp.float32(-jnp.inf)
    for c in range(M_BLK // CH):
        s = scores_ref[c * CH:(c + 1) * CH, :]
        i = jax.lax.bitcast_convert_type(s, jnp.int32)
        sgn = jax.lax.shift_right_arithmetic(i, 31)
        tie = cols63 ^ (sgn & m6)
        key = jax.lax.bitcast_convert_type((i & ~m6) | tie, jnp.float32)
        tops = []
        for _ in range(K):
            m = jnp.max(key, axis=1, keepdims=True)
            tops.append(m)
            key = jnp.where(key == m, neg_inf, key)
        tk = jnp.concatenate(tops, axis=1)
        tb = jax.lax.bitcast_convert_type(tk, jnp.int32)
        tsgn = jax.lax.shift_right_arithmetic(tb, 31)
        top_idx = (tb & m6) ^ (~tsgn & m6)
        vals = jax.lax.bitcast_convert_type(tb & ~m6, jnp.float32)
        e = jnp.exp(vals - vals[:, 0:1])
        probs = e / jnp.sum(e, axis=1, keepdims=True)
        probs_ref[c * CH:(c + 1) * CH, :] = probs
        idx_ref[c * CH:(c + 1) * CH, :] = top_idx


@jax.jit
def kernel(x, W_gate):
    grid = (NT // M_BLK,)
    probs, idx, scores = pl.pallas_call(
        _router_block,
        grid=grid,
        in_specs=[
            pl.BlockSpec((M_BLK, EMB), lambda i: (i, 0)),
            pl.BlockSpec((NE, EMB), lambda i: (0, 0)),
        ],
        out_specs=[
            pl.BlockSpec((M_BLK, K), lambda i: (i, 0)),
            pl.BlockSpec((M_BLK, K), lambda i: (i, 0)),
            pl.BlockSpec((M_BLK, NE), lambda i: (i, 0)),
        ],
        out_shape=[
            jax.ShapeDtypeStruct((NT, K), jnp.float32),
            jax.ShapeDtypeStruct((NT, K), jnp.int32),
            jax.ShapeDtypeStruct((NT, NE), jnp.float32),
        ],
    )(x, W_gate)
    return (probs, idx, scores)


# speedup vs baseline: 1.1644x; 1.0539x over previous
import jax
import jax.numpy as jnp
from jax.experimental import pallas as pl

EMB = 4096
NE = 64
K = 8
NT = 8192
M_BLK = 1024
CH = 64


def _router_block(x_ref, w_ref, probs_ref, idx_ref, scores_ref):
    x = x_ref[...]
    w = w_ref[...]
    scores = jax.lax.dot_general(
        x, w, (((1,), (1,)), ((), ())), preferred_element_type=jnp.float32
    )
    scores_ref[...] = scores

    cols63 = jnp.int32(NE - 1) - jax.lax.broadcasted_iota(jnp.int32, (CH, NE), 1)
    m6 = jnp.int32(NE - 1)
    neg_inf = jnp.float32(-jnp.inf)
    for c in range(M_BLK // CH):
        s = scores_ref[c * CH:(c + 1) * CH, :]
        i = jax.lax.bitcast_convert_type(s, jnp.int32)
        sgn = jax.lax.shift_right_arithmetic(i, 31)
        tie = cols63 ^ (sgn & m6)
        key = jax.lax.bitcast_convert_type((i & ~m6) | tie, jnp.float32)
        tops = []
        for _ in range(K):
            m = jnp.max(key, axis=1, keepdims=True)
            tops.append(m)
            key = jnp.where(key == m, neg_inf, key)
        tk = jnp.concatenate(tops, axis=1)
        tb = jax.lax.bitcast_convert_type(tk, jnp.int32)
        tsgn = jax.lax.shift_right_arithmetic(tb, 31)
        top_idx = (tb & m6) ^ (~tsgn & m6)
        vals = jax.lax.bitcast_convert_type(tb & ~m6, jnp.float32)
        e = jnp.exp(vals - vals[:, 0:1])
        probs = e / jnp.sum(e, axis=1, keepdims=True)
        probs_ref[c * CH:(c + 1) * CH, :] = probs
        idx_ref[c * CH:(c + 1) * CH, :] = top_idx


@jax.jit
def kernel(x, W_gate):
    grid = (NT // M_BLK,)
    probs, idx, scores = pl.pallas_call(
        _router_block,
        grid=grid,
        in_specs=[
            pl.BlockSpec((M_BLK, EMB), lambda i: (i, 0)),
            pl.BlockSpec((NE, EMB), lambda i: (0, 0)),
        ],
        out_specs=[
            pl.BlockSpec((M_BLK, K), lambda i: (i, 0)),
            pl.BlockSpec((M_BLK, K), lambda i: (i, 0)),
            pl.BlockSpec((M_BLK, NE), lambda i: (i, 0)),
        ],
        out_shape=[
            jax.ShapeDtypeStruct((NT, K), jnp.float32),
            jax.ShapeDtypeStruct((NT, K), jnp.int32),
            jax.ShapeDtypeStruct((NT, NE), jnp.float32),
        ],
    )(x, W_gate)
    return (probs, idx, scores)


# speedup vs baseline: 1.2222x; 1.0496x over previous
import jax
import jax.numpy as jnp
from jax.experimental import pallas as pl

EMB = 4096
NE = 64
K = 8
NT = 8192
M_BLK = 1024
SUB = 128


def _router_block(x_ref, w_ref, probs_ref, idx_ref, scores_ref):
    w = w_ref[...]
    cols63 = jnp.int32(NE - 1) - jax.lax.broadcasted_iota(jnp.int32, (SUB, NE), 1)
    m6 = jnp.int32(NE - 1)
    neg_inf = jnp.float32(-jnp.inf)
    for c in range(M_BLK // SUB):
        x = x_ref[c * SUB:(c + 1) * SUB, :]
        s = jax.lax.dot_general(
            x, w, (((1,), (1,)), ((), ())), preferred_element_type=jnp.float32
        )
        scores_ref[c * SUB:(c + 1) * SUB, :] = s
        i = jax.lax.bitcast_convert_type(s, jnp.int32)
        sgn = jax.lax.shift_right_arithmetic(i, 31)
        tie = cols63 ^ (sgn & m6)
        key = jax.lax.bitcast_convert_type((i & ~m6) | tie, jnp.float32)
        tops = []
        for _ in range(K):
            m = jnp.max(key, axis=1, keepdims=True)
            tops.append(m)
            key = jnp.where(key == m, neg_inf, key)
        tk = jnp.concatenate(tops, axis=1)
        tb = jax.lax.bitcast_convert_type(tk, jnp.int32)
        tsgn = jax.lax.shift_right_arithmetic(tb, 31)
        top_idx = (tb & m6) ^ (~tsgn & m6)
        vals = jax.lax.bitcast_convert_type(tb & ~m6, jnp.float32)
        e = jnp.exp(vals - vals[:, 0:1])
        probs = e / jnp.sum(e, axis=1, keepdims=True)
        probs_ref[c * SUB:(c + 1) * SUB, :] = probs
        idx_ref[c * SUB:(c + 1) * SUB, :] = top_idx


@jax.jit
def kernel(x, W_gate):
    grid = (NT // M_BLK,)
    probs, idx, scores = pl.pallas_call(
        _router_block,
        grid=grid,
        in_specs=[
            pl.BlockSpec((M_BLK, EMB), lambda i: (i, 0)),
            pl.BlockSpec((NE, EMB), lambda i: (0, 0)),
        ],
        out_specs=[
            pl.BlockSpec((M_BLK, K), lambda i: (i, 0)),
            pl.BlockSpec((M_BLK, K), lambda i: (i, 0)),
            pl.BlockSpec((M_BLK, NE), lambda i: (i, 0)),
        ],
        out_shape=[
            jax.ShapeDtypeStruct((NT, K), jnp.float32),
            jax.ShapeDtypeStruct((NT, K), jnp.int32),
            jax.ShapeDtypeStruct((NT, NE), jnp.float32),
        ],
    )(x, W_gate)
    return (probs, idx, scores)
